# baseline (speedup 1.0000x reference)
import jax
import jax.numpy as jnp
from jax.experimental import pallas as pl

_B_BLK = 32


def _pe_kernel(x_ref, pt_ref, o_ref):
    o_ref[...] = x_ref[...] + pt_ref[...][None, :, :]


def kernel(x, pos_table):
    batch, seq, d = x.shape
    grid = (batch // _B_BLK,)
    return pl.pallas_call(
        _pe_kernel,
        grid=grid,
        in_specs=[
            pl.BlockSpec((_B_BLK, seq, d), lambda i: (i, 0, 0)),
            pl.BlockSpec((seq, d), lambda i: (0, 0)),
        ],
        out_specs=pl.BlockSpec((_B_BLK, seq, d), lambda i: (i, 0, 0)),
        out_shape=jax.ShapeDtypeStruct((batch, seq, d), x.dtype),
    )(x, pos_table)


# speedup vs baseline: 1.0175x; 1.0175x over previous
import jax
import jax.numpy as jnp
from jax.experimental import pallas as pl

_B_BLK = 64


def _pe_kernel(x_ref, pt_ref, o_ref):
    o_ref[...] = x_ref[...] + pt_ref[...][None, :, :]


def kernel(x, pos_table):
    batch, seq, d = x.shape
    grid = (batch // _B_BLK,)
    return pl.pallas_call(
        _pe_kernel,
        grid=grid,
        in_specs=[
            pl.BlockSpec((_B_BLK, seq, d), lambda i: (i, 0, 0)),
            pl.BlockSpec((seq, d), lambda i: (0, 0)),
        ],
        out_specs=pl.BlockSpec((_B_BLK, seq, d), lambda i: (i, 0, 0)),
        out_shape=jax.ShapeDtypeStruct((batch, seq, d), x.dtype),
    )(x, pos_table)


# speedup vs baseline: 1.0263x; 1.0087x over previous
import jax
import jax.numpy as jnp
from jax.experimental import pallas as pl

_B_BLK = 128


def _pe_kernel(x_ref, pt_ref, o_ref):
    o_ref[...] = x_ref[...] + pt_ref[...][None, :, :]


def kernel(x, pos_table):
    batch, seq, d = x.shape
    grid = (batch // _B_BLK,)
    return pl.pallas_call(
        _pe_kernel,
        grid=grid,
        in_specs=[
            pl.BlockSpec((_B_BLK, seq, d), lambda i: (i, 0, 0)),
            pl.BlockSpec((seq, d), lambda i: (0, 0)),
        ],
        out_specs=pl.BlockSpec((_B_BLK, seq, d), lambda i: (i, 0, 0)),
        out_shape=jax.ShapeDtypeStruct((batch, seq, d), x.dtype),
    )(x, pos_table)
